# Initial kernel scaffold; baseline (speedup 1.0000x reference)
import jax
import jax.numpy as jnp
from jax import lax
from jax.experimental import pallas as pl
from jax.experimental.pallas import tpu as pltpu
from jax.experimental.pallas import tpu_sc as plsc

N_NODES = 50000
CHANNELS = 32
N_EDGES = 1600000

NC = 2
NS = 16
NW = NC * NS

BB = 128
GB = 25
GE = BB * GB
NGROUPS = N_EDGES // GE
GPW = (NGROUPS + NW - 1) // NW

ROWS_PER_TILE = N_NODES // NS
ZROWS = 625
CCHUNK = 2000
NCCHUNK = N_NODES // CCHUNK


def _sc_body(x_hbm, src_hbm, tgt_hbm, psums_hbm, pcnt_hbm,
             src_v, tgt_v, rows_v, ones_v, zc_v,
             sums_sh, cnt_sh, sem):
    c = lax.axis_index("c")
    s = lax.axis_index("s")
    w = s * NC + c

    z16 = jnp.zeros((16,), jnp.float32)
    o16 = jnp.full((16,), 1.0, jnp.float32)

    def _zrow(i, carry):
        rows_v[i, pl.ds(0, 16)] = z16
        rows_v[i, pl.ds(16, 16)] = z16
        return carry
    lax.fori_loop(0, ZROWS, _zrow, 0)

    def _zc(i, carry):
        zc_v[pl.ds(i * 16, 16)] = z16
        return carry
    lax.fori_loop(0, CCHUNK // 16, _zc, 0)

    def _ofill(i, carry):
        ones_v[pl.ds(i * 16, 16)] = o16
        return carry
    lax.fori_loop(0, BB // 16, _ofill, 0)

    for k in range(ROWS_PER_TILE // ZROWS):
        pltpu.sync_copy(rows_v.at[pl.ds(0, ZROWS), :],
                        sums_sh.at[pl.ds(s * ROWS_PER_TILE + k * ZROWS, ZROWS), :])
    pltpu.sync_copy(zc_v, cnt_sh.at[pl.ds(s * CCHUNK, CCHUNK)])

    @pl.when(s < NCCHUNK - NS)
    def _():
        pltpu.sync_copy(zc_v, cnt_sh.at[pl.ds((NS + s) * CCHUNK, CCHUNK)])

    plsc.subcore_barrier()

    def _group(k, carry):
        g = w + k * NW

        @pl.when(g < NGROUPS)
        def _():
            pltpu.sync_copy(src_hbm.at[pl.ds(g * GB, GB), :], src_v)
            pltpu.sync_copy(tgt_hbm.at[pl.ds(g * GB, GB), :], tgt_v)

            def _batch(j, carry2):
                rslice = rows_v.at[pl.ds(j * BB, BB), :]
                pltpu.async_copy(x_hbm.at[src_v.at[j]], rslice, sem).wait()
                pltpu.sync_copy(rslice, sums_sh.at[tgt_v.at[j]], add=True)
                pltpu.sync_copy(ones_v, cnt_sh.at[tgt_v.at[j]], add=True)
                return carry2
            lax.fori_loop(0, GB, _batch, 0)
        return carry
    lax.fori_loop(0, GPW, _group, 0)

    plsc.subcore_barrier()

    for k in range(ROWS_PER_TILE // ZROWS):
        r0 = s * ROWS_PER_TILE + k * ZROWS
        pltpu.sync_copy(sums_sh.at[pl.ds(r0, ZROWS), :], rows_v.at[pl.ds(0, ZROWS), :])
        pltpu.sync_copy(rows_v.at[pl.ds(0, ZROWS), :], psums_hbm.at[c, pl.ds(r0, ZROWS), :])

    pltpu.sync_copy(cnt_sh.at[pl.ds(s * CCHUNK, CCHUNK)], zc_v)
    pltpu.sync_copy(zc_v, pcnt_hbm.at[c, pl.ds(s * CCHUNK, CCHUNK)])

    @pl.when(s < NCCHUNK - NS)
    def _():
        m0 = (NS + s) * CCHUNK
        pltpu.sync_copy(cnt_sh.at[pl.ds(m0, CCHUNK)], zc_v)
        pltpu.sync_copy(zc_v, pcnt_hbm.at[c, pl.ds(m0, CCHUNK)])


_sc_scatter = pl.kernel(
    _sc_body,
    out_type=(
        jax.ShapeDtypeStruct((NC, N_NODES, CHANNELS), jnp.float32),
        jax.ShapeDtypeStruct((NC, N_NODES), jnp.float32),
    ),
    mesh=plsc.VectorSubcoreMesh(core_axis_name="c", subcore_axis_name="s"),
    scratch_types=[
        pltpu.VMEM((GB, BB), jnp.int32),
        pltpu.VMEM((GB, BB), jnp.int32),
        pltpu.VMEM((GE, CHANNELS), jnp.float32),
        pltpu.VMEM((BB,), jnp.float32),
        pltpu.VMEM((CCHUNK,), jnp.float32),
        pltpu.VMEM_SHARED((N_NODES, CHANNELS), jnp.float32),
        pltpu.VMEM_SHARED((N_NODES,), jnp.float32),
        pltpu.SemaphoreType.DMA,
    ],
)


def _tc_body(p_ref, cnt_ref, gamma_ref, beta_ref, w_ref, b_ref, out_ref):
    ssum = p_ref[0] + p_ref[1]
    cnt = cnt_ref[0] + cnt_ref[1]
    agg = ssum / jnp.maximum(cnt, 1.0)
    mu = jnp.mean(agg, axis=0, keepdims=True)
    d = agg - mu
    var = jnp.mean(d * d, axis=0, keepdims=True)
    scale = gamma_ref[...] * lax.rsqrt(var + 1e-5)
    xb = d * scale + beta_ref[...]
    y = lax.dot_general(xb, w_ref[...], (((1,), (1,)), ((), ())),
                        preferred_element_type=jnp.float32)
    out_ref[...] = jnp.maximum(y + b_ref[...], 0.0)


def kernel(x, sources, targets, gamma, beta, W, b):
    src2 = jnp.asarray(sources, jnp.int32).reshape(N_EDGES // BB, BB)
    tgt2 = jnp.asarray(targets, jnp.int32).reshape(N_EDGES // BB, BB)
    psums, pcnt = _sc_scatter(x, src2, tgt2)
    pcnt3 = pcnt.reshape(NC, N_NODES, 1)
    out = pl.pallas_call(
        _tc_body,
        out_shape=jax.ShapeDtypeStruct((N_NODES, CHANNELS), jnp.float32),
    )(psums, pcnt3, gamma.reshape(1, CHANNELS), beta.reshape(1, CHANNELS),
      W, b.reshape(1, CHANNELS))
    return out


# speedup vs baseline: 24.6705x; 24.6705x over previous
import jax
import jax.numpy as jnp
from jax import lax
from jax.experimental import pallas as pl
from jax.experimental.pallas import tpu as pltpu
from jax.experimental.pallas import tpu_sc as plsc

N_NODES = 50000
CHANNELS = 32
N_EDGES = 1600000

NC = 2
NS = 16

HALF = N_NODES // NC
TRASH = 2048
N_ACC = HALF + TRASH

BB = 128
BPT = 800
GB = 40
NGRP = BPT // GB
NB = NS * BPT
E_PAD = NB * BB

NBUF = 4

ZROWS = 1000
NZCH = HALF // ZROWS
CCHUNK = 1000
NCCH = HALF // CCHUNK


def _sc_body(x_hbm, src_hbm, tgt_hbm, psums_hbm, pcnt_hbm,
             src_v, tgt_v, rows_v, zrows_v, ones_v, zc_v,
             sums_sh, cnt_sh,
             gsem0, gsem1, gsem2, gsem3, ssem0, ssem1, ssem2, ssem3, isem):
    gsems = (gsem0, gsem1, gsem2, gsem3)
    ssems = (ssem0, ssem1, ssem2, ssem3)
    c = lax.axis_index("c")
    s = lax.axis_index("s")

    z16 = jnp.zeros((16,), jnp.float32)
    o16 = jnp.full((16,), 1.0, jnp.float32)

    def _zrow(i, carry):
        zrows_v[i, pl.ds(0, 16)] = z16
        zrows_v[i, pl.ds(16, 16)] = z16
        return carry
    lax.fori_loop(0, ZROWS, _zrow, 0)

    def _zc(i, carry):
        zc_v[pl.ds(i * 16, 16)] = z16
        return carry
    lax.fori_loop(0, 1008 // 16, _zc, 0)

    def _ofill(i, carry):
        ones_v[pl.ds(i * 16, 16)] = o16
        return carry
    lax.fori_loop(0, BB // 16, _ofill, 0)

    def _zero_chunk(m):
        pltpu.sync_copy(zrows_v, sums_sh.at[pl.ds(m * ZROWS, ZROWS), :])
        pltpu.sync_copy(zc_v.at[pl.ds(0, CCHUNK)],
                        cnt_sh.at[pl.ds(m * CCHUNK, CCHUNK)])

    _zero_chunk(s)

    @pl.when(s < NZCH - NS)
    def _():
        _zero_chunk(NS + s)

    plsc.subcore_barrier()

    base = c * HALF
    tile_b0 = s * BPT

    def _stage_idx(g, sem):
        pltpu.async_copy(src_hbm.at[pl.ds(tile_b0 + g * GB, GB), :],
                         src_v.at[g % 2], sem)
        pltpu.async_copy(tgt_hbm.at[pl.ds(tile_b0 + g * GB, GB), :],
                         tgt_v.at[g % 2], sem)

    def _wait_idx():
        pltpu.make_async_copy(src_hbm.at[pl.ds(0, GB), :], src_v.at[0], isem).wait()
        pltpu.make_async_copy(tgt_hbm.at[pl.ds(0, GB), :], tgt_v.at[0], isem).wait()

    def _fire_gather(n, b):
        g = n // GB
        pltpu.async_copy(x_hbm.at[src_v.at[g % 2, n % GB]], rows_v.at[b], gsems[b])

    def _remap_and_fire_scatter(n, b):
        g = n // GB
        r = n % GB
        ib = g % 2
        for cl in range(BB // 16):
            t = tgt_v[ib, r, pl.ds(cl * 16, 16)]
            tl = t - base
            inr = jnp.logical_and(tl >= 0, tl < HALF)
            trash = HALF + jnp.bitwise_and(t, TRASH - 1)
            tgt_v[ib, r, pl.ds(cl * 16, 16)] = jnp.where(inr, tl, trash)
        idxr = tgt_v.at[ib, r]
        pltpu.async_copy(rows_v.at[b], sums_sh.at[idxr], ssems[b], add=True)
        pltpu.async_copy(ones_v, cnt_sh.at[idxr], ssems[b], add=True)

    def _drain_gather(b):
        pltpu.make_async_copy(x_hbm.at[pl.ds(0, BB), :], rows_v.at[b],
                              gsems[b]).wait()

    def _drain_scatter(b):
        pltpu.make_async_copy(rows_v.at[b], sums_sh.at[pl.ds(0, BB), :],
                              ssems[b]).wait()
        pltpu.make_async_copy(ones_v, cnt_sh.at[pl.ds(0, BB)],
                              ssems[b]).wait()

    _stage_idx(0, isem)
    _wait_idx()
    _stage_idx(1, isem)
    _fire_gather(0, 0)
    _fire_gather(1, 1)

    def _step(i, carry):
        for b in range(NBUF):
            n = i * NBUF + b
            if b < 2:
                @pl.when(i > 0)
                def _():
                    _drain_scatter(b + 2)
            else:
                _drain_scatter(b - 2)

            if b == 2:
                @pl.when(jnp.logical_and(i % 10 == 9, i < BPT // NBUF - 1))
                def _():
                    _wait_idx()

                @pl.when(jnp.logical_and(i % 10 == 4,
                                         jnp.logical_and(i >= 14, i < 190)))
                def _():
                    _stage_idx((i * NBUF + 4) // GB + 1, isem)

            if b < 2:
                _fire_gather(n + 2, (b + 2) % NBUF)
            else:
                @pl.when(i < BPT // NBUF - 1)
                def _():
                    _fire_gather(n + 2, (b + 2) % NBUF)

            _drain_gather(b)
            _remap_and_fire_scatter(n, b)
        return carry

    lax.fori_loop(0, BPT // NBUF, _step, 0)

    _drain_scatter(2)
    _drain_scatter(3)

    plsc.subcore_barrier()

    def _copy_chunk(m):
        pltpu.sync_copy(sums_sh.at[pl.ds(m * ZROWS, ZROWS), :], zrows_v)
        pltpu.sync_copy(zrows_v, psums_hbm.at[pl.ds(c * HALF + m * ZROWS, ZROWS), :])
        pltpu.sync_copy(cnt_sh.at[pl.ds(m * CCHUNK, CCHUNK)], zc_v.at[pl.ds(0, CCHUNK)])
        pltpu.sync_copy(zc_v.at[pl.ds(0, CCHUNK)],
                        pcnt_hbm.at[pl.ds(c * HALF + m * CCHUNK, CCHUNK)])

    _copy_chunk(s)

    @pl.when(s < NZCH - NS)
    def _():
        _copy_chunk(NS + s)


_sc_scatter = pl.kernel(
    _sc_body,
    out_type=(
        jax.ShapeDtypeStruct((N_NODES, CHANNELS), jnp.float32),
        jax.ShapeDtypeStruct((N_NODES,), jnp.float32),
    ),
    mesh=plsc.VectorSubcoreMesh(core_axis_name="c", subcore_axis_name="s"),
    compiler_params=pltpu.CompilerParams(use_tc_tiling_on_sc=False),
    scratch_types=[
        pltpu.VMEM((2, GB, BB), jnp.int32),
        pltpu.VMEM((2, GB, BB), jnp.int32),
        pltpu.VMEM((NBUF, BB, CHANNELS), jnp.float32),
        pltpu.VMEM((ZROWS, CHANNELS), jnp.float32),
        pltpu.VMEM((BB,), jnp.float32),
        pltpu.VMEM((1008,), jnp.float32),
        pltpu.VMEM_SHARED((N_ACC, CHANNELS), jnp.float32),
        pltpu.VMEM_SHARED((N_ACC,), jnp.float32),
        pltpu.SemaphoreType.DMA,
        pltpu.SemaphoreType.DMA,
        pltpu.SemaphoreType.DMA,
        pltpu.SemaphoreType.DMA,
        pltpu.SemaphoreType.DMA,
        pltpu.SemaphoreType.DMA,
        pltpu.SemaphoreType.DMA,
        pltpu.SemaphoreType.DMA,
        pltpu.SemaphoreType.DMA,
    ],
)


TC_BLK = 5000
TC_GRID = N_NODES // TC_BLK


def _stats_body(sums_ref, cnt_ref, acc_ref):
    i = pl.program_id(0)
    agg = sums_ref[...] / jnp.maximum(cnt_ref[...], 1.0)
    ps = jnp.sum(agg, axis=0, keepdims=True)
    pq = jnp.sum(agg * agg, axis=0, keepdims=True)
    blk = jnp.concatenate([ps, pq], axis=0)

    @pl.when(i == 0)
    def _():
        acc_ref[...] = blk

    @pl.when(i > 0)
    def _():
        acc_ref[...] += blk


def _out_body(sums_ref, cnt_ref, stats_ref, gamma_ref, beta_ref, w_ref, b_ref,
              out_ref):
    agg = sums_ref[...] / jnp.maximum(cnt_ref[...], 1.0)
    mu = stats_ref[0:1, :] * (1.0 / N_NODES)
    var = stats_ref[1:2, :] * (1.0 / N_NODES) - mu * mu
    scale = gamma_ref[...] * lax.rsqrt(var + 1e-5)
    shift = beta_ref[...] - mu * scale
    xb = agg * scale + shift
    y = lax.dot_general(xb, w_ref[...], (((1,), (1,)), ((), ())),
                        preferred_element_type=jnp.float32)
    out_ref[...] = jnp.maximum(y + b_ref[...], 0.0)


def kernel(x, sources, targets, gamma, beta, W, b):
    pad = E_PAD - N_EDGES
    pad_iota = jnp.arange(pad, dtype=jnp.int32)
    src_pad = pad_iota % N_NODES
    tgt_pad = N_NODES + (pad_iota % TRASH)
    src2 = jnp.concatenate([jnp.asarray(sources, jnp.int32), src_pad]).reshape(NB, BB)
    tgt2 = jnp.concatenate([jnp.asarray(targets, jnp.int32), tgt_pad]).reshape(NB, BB)
    sums, cnt = _sc_scatter(x, src2, tgt2)
    cnt2 = cnt.reshape(N_NODES, 1)
    row_blk = lambda i: (i, 0)
    rep_blk = lambda i: (0, 0)
    stats = pl.pallas_call(
        _stats_body,
        grid=(TC_GRID,),
        in_specs=[
            pl.BlockSpec((TC_BLK, CHANNELS), row_blk),
            pl.BlockSpec((TC_BLK, 1), row_blk),
        ],
        out_specs=pl.BlockSpec((2, CHANNELS), rep_blk),
        out_shape=jax.ShapeDtypeStruct((2, CHANNELS), jnp.float32),
    )(sums, cnt2)
    out = pl.pallas_call(
        _out_body,
        grid=(TC_GRID,),
        in_specs=[
            pl.BlockSpec((TC_BLK, CHANNELS), row_blk),
            pl.BlockSpec((TC_BLK, 1), row_blk),
            pl.BlockSpec((2, CHANNELS), rep_blk),
            pl.BlockSpec((1, CHANNELS), rep_blk),
            pl.BlockSpec((1, CHANNELS), rep_blk),
            pl.BlockSpec((CHANNELS, CHANNELS), rep_blk),
            pl.BlockSpec((1, CHANNELS), rep_blk),
        ],
        out_specs=pl.BlockSpec((TC_BLK, CHANNELS), row_blk),
        out_shape=jax.ShapeDtypeStruct((N_NODES, CHANNELS), jnp.float32),
    )(sums, cnt2, stats, gamma.reshape(1, CHANNELS), beta.reshape(1, CHANNELS),
      W, b.reshape(1, CHANNELS))
    return out


# speedup vs baseline: 31.9574x; 1.2954x over previous
import jax
import jax.numpy as jnp
from jax import lax
from jax.experimental import pallas as pl
from jax.experimental.pallas import tpu as pltpu
from jax.experimental.pallas import tpu_sc as plsc

N_NODES = 50000
CHANNELS = 32
N_EDGES = 1600000

NC = 2
NS = 16

HALF = N_NODES // NC
TRASH = 512
N_ACC = HALF + TRASH

BB = 128
BPT = 800
GB = 40
NGRP = BPT // GB
NB = NS * BPT
E_PAD = NB * BB

NBUF = 8
SDEPTH = 4

CCAP = GB * BB + 256

ZROWS = 1000
NZCH = HALF // ZROWS
CCHUNK = 1000
NCCH = HALF // CCHUNK


def _sc_body(x_hbm, src_hbm, tgt_hbm, psums_hbm, pcnt_hbm,
             src_v, tgt_v, rows_v, srcc_v, tgtc_v, srci_v, tgti_v,
             ones_v, zc_v, sums_sh, cnt_sh, gsem, ssem, isem):
    c = lax.axis_index("c")
    s = lax.axis_index("s")

    z16 = jnp.zeros((16,), jnp.float32)
    o16 = jnp.full((16,), 1.0, jnp.float32)

    def _zrow(i, carry):
        rows_v[i, pl.ds(0, 16)] = z16
        rows_v[i, pl.ds(16, 16)] = z16
        return carry
    lax.fori_loop(0, ZROWS, _zrow, 0)

    def _zc(i, carry):
        zc_v[pl.ds(i * 16, 16)] = z16
        return carry
    lax.fori_loop(0, 1008 // 16, _zc, 0)

    def _ofill(i, carry):
        ones_v[pl.ds(i * 16, 16)] = o16
        return carry
    lax.fori_loop(0, BB // 16, _ofill, 0)

    def _zero_chunk(m):
        pltpu.sync_copy(rows_v.at[pl.ds(0, ZROWS), :],
                        sums_sh.at[pl.ds(m * ZROWS, ZROWS), :])
        pltpu.sync_copy(zc_v.at[pl.ds(0, CCHUNK)],
                        cnt_sh.at[pl.ds(m * CCHUNK, CCHUNK)])

    _zero_chunk(s)

    @pl.when(s < NZCH - NS)
    def _():
        _zero_chunk(NS + s)

    plsc.subcore_barrier()

    base = c * HALF
    tile_b0 = s * BPT

    def _stage_idx(g):
        pltpu.async_copy(src_hbm.at[pl.ds(tile_b0 + g * GB, GB), :],
                         src_v.at[g % 2], isem)
        pltpu.async_copy(tgt_hbm.at[pl.ds(tile_b0 + g * GB, GB), :],
                         tgt_v.at[g % 2], isem)

    def _wait_idx():
        pltpu.make_async_copy(src_hbm.at[pl.ds(0, GB), :], src_v.at[0], isem).wait()
        pltpu.make_async_copy(tgt_hbm.at[pl.ds(0, GB), :], tgt_v.at[0], isem).wait()

    def _drain_g1():
        pltpu.make_async_copy(x_hbm.at[pl.ds(0, BB), :],
                              rows_v.at[pl.ds(0, BB), :], gsem).wait()

    def _drain_s1():
        pltpu.make_async_copy(rows_v.at[pl.ds(0, BB), :],
                              sums_sh.at[pl.ds(0, BB), :], ssem).wait()
        pltpu.make_async_copy(ones_v, cnt_sh.at[pl.ds(0, BB)], ssem).wait()

    def _fire_gather_block(kf, J):
        slot = J % NBUF
        o = kf * BB
        for q in range(BB // 16):
            srci_v[slot, pl.ds(q * 16, 16)] = srcc_v[pl.ds(o + q * 16, 16)]
            tgti_v[slot, pl.ds(q * 16, 16)] = tgtc_v[pl.ds(o + q * 16, 16)]
        rbuf = rows_v.at[pl.ds(slot * BB, BB), :]
        pltpu.async_copy(x_hbm.at[srci_v.at[slot]], rbuf, gsem)

    def _fire_scatter_block(j):
        slot = j % NBUF
        rbuf = rows_v.at[pl.ds(slot * BB, BB), :]
        idxr = tgti_v.at[slot]
        pltpu.async_copy(rbuf, sums_sh.at[idxr], ssem, add=True)
        pltpu.async_copy(ones_v, cnt_sh.at[idxr], ssem, add=True)

    _stage_idx(0)
    _wait_idx()

    def _group(g, carry):
        W, J, sf, sd = carry
        par = g % 2

        @pl.when(g + 1 < NGRP)
        def _():
            _stage_idx(g + 1)

        @pl.when(g > 0)
        def _():
            _wait_idx()

        def _crow(r, cr):
            W1, kf1, J1, sf1, sd1 = cr
            tls, svs, masks, offs = [], [], [], []
            off = W1
            for cl in range(BB // 16):
                t = tgt_v[par, r, pl.ds(cl * 16, 16)]
                sv = src_v[par, r, pl.ds(cl * 16, 16)]
                tl = t - base
                inr = plsc.bitcast(tl, jnp.uint32) < jnp.uint32(HALF)
                tls.append(tl)
                svs.append(sv)
                masks.append(inr)
            pcs = [jnp.max(plsc.all_reduce_population_count(m)) for m in masks]
            for cl in range(BB // 16):
                offs.append(off)
                off = off + pcs[cl]
            for cl in range(BB // 16):
                plsc.store_compressed(tgtc_v.at[pl.ds(offs[cl], 16)],
                                      tls[cl], mask=masks[cl])
                plsc.store_compressed(srcc_v.at[pl.ds(offs[cl], 16)],
                                      svs[cl], mask=masks[cl])
            W1 = off

            fire = (W1 // BB > kf1).astype(jnp.int32)
            scat = jnp.logical_and(fire == 1, kf1 >= 2).astype(jnp.int32)
            drn = jnp.logical_and(fire == 1, sf1 - sd1 >= SDEPTH).astype(jnp.int32)

            @pl.when(fire == 1)
            def _():
                @pl.when(sf1 - sd1 >= SDEPTH)
                def _():
                    _drain_s1()
                _fire_gather_block(kf1, J1)

                @pl.when(kf1 >= 2)
                def _():
                    _drain_g1()
                    _fire_scatter_block(J1 - 2)

            return (W1, kf1 + fire, J1 + fire, sf1 + scat, sd1 + drn)

        W, kf, J, sf, sd = lax.fori_loop(0, GB, _crow, (W, 0, J, sf, sd))
        nf = W // BB

        def _tail(k, cr):
            J1, sf1, sd1 = cr
            drn = (sf1 - sd1 >= SDEPTH).astype(jnp.int32)

            @pl.when(sf1 - sd1 >= SDEPTH)
            def _():
                _drain_s1()
            _drain_g1()
            _fire_scatter_block(J1 - nf + k)
            return (J1, sf1 + 1, sd1 + drn)

        J, sf, sd = lax.fori_loop(jnp.maximum(nf - 2, 0), nf, _tail, (J, sf, sd))

        L = W - nf * BB
        o = nf * BB

        @pl.when(L > 0)
        def _():
            for q in range(BB // 16):
                srcc_v[pl.ds(q * 16, 16)] = srcc_v[pl.ds(o + q * 16, 16)]
                tgtc_v[pl.ds(q * 16, 16)] = tgtc_v[pl.ds(o + q * 16, 16)]

        return (L, J, sf, sd)

    W, J, sf, sd = lax.fori_loop(0, NGRP, _group, (0, 0, 0, 0))

    iota16 = lax.iota(jnp.int32, 16)

    @pl.when(W > 0)
    def _():
        for q in range(BB // 16):
            pad_src = iota16 + (q * 16)
            pad_tgt = pad_src + HALF
            srcc_v[pl.ds(W + q * 16, 16)] = pad_src
            tgtc_v[pl.ds(W + q * 16, 16)] = pad_tgt
        _fire_gather_block(0, J)
        _drain_g1()
        _fire_scatter_block(J)

    sf = sf + (W > 0).astype(jnp.int32)

    def _draina(i, carry):
        _drain_s1()
        return carry
    lax.fori_loop(sd, sf, _draina, 0)

    plsc.subcore_barrier()

    def _copy_chunk(m):
        pltpu.sync_copy(sums_sh.at[pl.ds(m * ZROWS, ZROWS), :],
                        rows_v.at[pl.ds(0, ZROWS), :])
        pltpu.sync_copy(rows_v.at[pl.ds(0, ZROWS), :],
                        psums_hbm.at[pl.ds(c * HALF + m * ZROWS, ZROWS), :])
        pltpu.sync_copy(cnt_sh.at[pl.ds(m * CCHUNK, CCHUNK)], zc_v.at[pl.ds(0, CCHUNK)])
        pltpu.sync_copy(zc_v.at[pl.ds(0, CCHUNK)],
                        pcnt_hbm.at[pl.ds(c * HALF + m * CCHUNK, CCHUNK)])

    _copy_chunk(s)

    @pl.when(s < NZCH - NS)
    def _():
        _copy_chunk(NS + s)


_sc_scatter = pl.kernel(
    _sc_body,
    out_type=(
        jax.ShapeDtypeStruct((N_NODES, CHANNELS), jnp.float32),
        jax.ShapeDtypeStruct((N_NODES,), jnp.float32),
    ),
    mesh=plsc.VectorSubcoreMesh(core_axis_name="c", subcore_axis_name="s"),
    compiler_params=pltpu.CompilerParams(use_tc_tiling_on_sc=False,
                                        needs_layout_passes=False),
    scratch_types=[
        pltpu.VMEM((2, GB, BB), jnp.int32),
        pltpu.VMEM((2, GB, BB), jnp.int32),
        pltpu.VMEM((NBUF * BB, CHANNELS), jnp.float32),
        pltpu.VMEM((CCAP,), jnp.int32),
        pltpu.VMEM((CCAP,), jnp.int32),
        pltpu.VMEM((NBUF, BB), jnp.int32),
        pltpu.VMEM((NBUF, BB), jnp.int32),
        pltpu.VMEM((BB,), jnp.float32),
        pltpu.VMEM((1008,), jnp.float32),
        pltpu.VMEM_SHARED((N_ACC, CHANNELS), jnp.float32),
        pltpu.VMEM_SHARED((N_ACC,), jnp.float32),
        pltpu.SemaphoreType.DMA,
        pltpu.SemaphoreType.DMA,
        pltpu.SemaphoreType.DMA,
    ],
)


TC_BLK = 5000
TC_GRID = N_NODES // TC_BLK


def _tc_body(sums_ref, cnt_ref, gamma_ref, beta_ref, w_ref, b_ref, out_ref,
             acc_ref):
    i = pl.program_id(0)
    agg = sums_ref[...] / jnp.maximum(cnt_ref[...], 1.0)

    @pl.when(i < TC_GRID)
    def _():
        ps = jnp.sum(agg, axis=0, keepdims=True)
        pq = jnp.sum(agg * agg, axis=0, keepdims=True)
        blk = jnp.concatenate([ps, pq], axis=0)

        @pl.when(i == 0)
        def _():
            acc_ref[...] = blk

        @pl.when(i > 0)
        def _():
            acc_ref[...] += blk

    @pl.when(i >= TC_GRID)
    def _():
        stats = acc_ref[...]
        mu = stats[0:1, :] * (1.0 / N_NODES)
        var = stats[1:2, :] * (1.0 / N_NODES) - mu * mu
        scale = gamma_ref[...] * lax.rsqrt(var + 1e-5)
        shift = beta_ref[...] - mu * scale
        xb = agg * scale + shift
        y = lax.dot_general(xb, w_ref[...], (((1,), (1,)), ((), ())),
                            preferred_element_type=jnp.float32)
        out_ref[...] = jnp.maximum(y + b_ref[...], 0.0)


def kernel(x, sources, targets, gamma, beta, W, b):
    pad = E_PAD - N_EDGES
    pad_iota = jnp.arange(pad, dtype=jnp.int32)
    src_pad = pad_iota % N_NODES
    tgt_pad = jnp.full((pad,), N_NODES, jnp.int32)
    src2 = jnp.concatenate([jnp.asarray(sources, jnp.int32), src_pad]).reshape(NB, BB)
    tgt2 = jnp.concatenate([jnp.asarray(targets, jnp.int32), tgt_pad]).reshape(NB, BB)
    sums, cnt = _sc_scatter(x, src2, tgt2)
    cnt2 = cnt.reshape(N_NODES, 1)
    row_blk = lambda i: (i % TC_GRID, 0)
    rep_blk = lambda i: (0, 0)
    out = pl.pallas_call(
        _tc_body,
        grid=(2 * TC_GRID,),
        in_specs=[
            pl.BlockSpec((TC_BLK, CHANNELS), row_blk),
            pl.BlockSpec((TC_BLK, 1), row_blk),
            pl.BlockSpec((1, CHANNELS), rep_blk),
            pl.BlockSpec((1, CHANNELS), rep_blk),
            pl.BlockSpec((CHANNELS, CHANNELS), rep_blk),
            pl.BlockSpec((1, CHANNELS), rep_blk),
        ],
        out_specs=pl.BlockSpec((TC_BLK, CHANNELS), row_blk),
        out_shape=jax.ShapeDtypeStruct((N_NODES, CHANNELS), jnp.float32),
        scratch_shapes=[pltpu.VMEM((2, CHANNELS), jnp.float32)],
    )(sums, cnt2, gamma.reshape(1, CHANNELS), beta.reshape(1, CHANNELS),
      W, b.reshape(1, CHANNELS))
    return out
